# Initial kernel scaffold; baseline (speedup 1.0000x reference)
import jax
import jax.numpy as jnp
from jax.experimental import pallas as pl


def kernel(backbone_tokens, params, num_masked_tracks):
    raise NotImplementedError("write your pallas kernel here")


# speedup vs baseline: 1.8248x; 1.8248x over previous
import functools
import math

import jax
import jax.numpy as jnp
from jax.experimental import pallas as pl
from jax.experimental.pallas import tpu as pltpu

B = 8; CB = 256; M = 2048; D = 128; NH = 4; L = 2; FF = 512; NOUT = 7
MAXQ = 1200; NMASK = 1120
DH = D // NH
_INV_SQRT_DH = 1.0 / math.sqrt(DH)

_W_NAMES = (
    'emb_sel', 'qn_g', 'qn_b', 'mn_g', 'mn_b', 'proj_W', 'proj_b',
    'sa_Wq', 'sa_bq', 'sa_Wk', 'sa_bk', 'sa_Wv', 'sa_bv', 'sa_Wo', 'sa_bo',
    'ca_Wq', 'ca_bq', 'ca_Wk', 'ca_bk', 'ca_Wv', 'ca_bv', 'ca_Wo', 'ca_bo',
    'n1_g', 'n1_b', 'n2_g', 'n2_b', 'n3_g', 'n3_b',
    'ff_W1', 'ff_b1', 'ff_W2', 'ff_b2',
    'out_W1', 'out_b1', 'out_W2p', 'out_b2p',
)
_W_IDX = {n: i for i, n in enumerate(_W_NAMES)}


def _gelu(x):
    return 0.5 * x * (1.0 + jax.lax.erf(x * (1.0 / math.sqrt(2.0))))


def _ln(x, g, b):
    mu = x.mean(-1, keepdims=True)
    var = ((x - mu) ** 2).mean(-1, keepdims=True)
    return (x - mu) * jax.lax.rsqrt(var + 1e-5) * g + b


def _mm(a, b):
    return jax.lax.dot_general(a, b, (((1,), (0,)), ((), ())),
                               preferred_element_type=jnp.float32)


def _mm_tr(a, b):
    return jax.lax.dot_general(a, b, (((1,), (1,)), ((), ())),
                               preferred_element_type=jnp.float32)


def _attn(q_in, kv, Wq, bq, Wk, bk, Wv, bv, Wo, bo):
    q = _mm(q_in, Wq) + bq
    k = _mm(kv, Wk) + bk
    v = _mm(kv, Wv) + bv
    outs = []
    for h in range(NH):
        sl = slice(h * DH, (h + 1) * DH)
        s = _mm_tr(q[:, sl], k[:, sl]) * _INV_SQRT_DH
        s = s - jnp.max(s, axis=-1, keepdims=True)
        p = jnp.exp(s)
        p = p / jnp.sum(p, axis=-1, keepdims=True)
        outs.append(_mm(p, v[:, sl]))
    o = jnp.concatenate(outs, axis=-1)
    return _mm(o, Wo) + bo


def _fwd_kernel(bb_ref, *refs):
    out_ref = refs[-1]
    w = lambda n: refs[_W_IDX[n]][...]

    bb = bb_ref[0]
    mem = jax.lax.dot_general(bb, w('proj_W'), (((0,), (0,)), ((), ())),
                              preferred_element_type=jnp.float32)
    mem = _ln(mem + w('proj_b'), w('mn_g'), w('mn_b'))

    x = _ln(w('emb_sel'), w('qn_g'), w('qn_b'))
    for l in range(L):
        h = _ln(x, w('n1_g')[l], w('n1_b')[l])
        x = x + _attn(h, h,
                      w('sa_Wq')[l], w('sa_bq')[l], w('sa_Wk')[l], w('sa_bk')[l],
                      w('sa_Wv')[l], w('sa_bv')[l], w('sa_Wo')[l], w('sa_bo')[l])
        h = _ln(x, w('n2_g')[l], w('n2_b')[l])
        x = x + _attn(h, mem,
                      w('ca_Wq')[l], w('ca_bq')[l], w('ca_Wk')[l], w('ca_bk')[l],
                      w('ca_Wv')[l], w('ca_bv')[l], w('ca_Wo')[l], w('ca_bo')[l])
        h = _ln(x, w('n3_g')[l], w('n3_b')[l])
        x = x + _mm(_gelu(_mm(h, w('ff_W1')[l]) + w('ff_b1')[l]), w('ff_W2')[l]) + w('ff_b2')[l]

    out = _mm(_gelu(_mm(x, w('out_W1')) + w('out_b1')),
              w('out_W2p')) + w('out_b2p')
    out_ref[0] = out


@jax.jit
def _run(backbone_tokens, params, num_masked_tracks):
    p = params
    emb_sel = jax.lax.dynamic_slice_in_dim(
        p['emb'], num_masked_tracks - NMASK, NMASK, axis=0)
    out_W2p = jnp.zeros((D, 128), jnp.float32).at[:, :NOUT].set(p['out_W2'])
    out_b2p = jnp.zeros((128,), jnp.float32).at[:NOUT].set(p['out_b2'])
    vals = {
        'emb_sel': emb_sel, 'out_W2p': out_W2p, 'out_b2p': out_b2p,
    }
    for n in _W_NAMES:
        if n not in vals:
            vals[n] = p[n]
    ops = [vals[n] for n in _W_NAMES]

    full = lambda a: pl.BlockSpec(a.shape, lambda b: (0,) * a.ndim)
    out = pl.pallas_call(
        _fwd_kernel,
        grid=(B,),
        in_specs=[pl.BlockSpec((1, CB, M), lambda b: (b, 0, 0))] +
                 [full(a) for a in ops],
        out_specs=pl.BlockSpec((1, NMASK, 128), lambda b: (b, 0, 0)),
        out_shape=jax.ShapeDtypeStruct((B, NMASK, 128), jnp.float32),
        compiler_params=pltpu.CompilerParams(
            dimension_semantics=("arbitrary",),
        ),
    )(backbone_tokens, *ops)
    return out[..., :NOUT].transpose(0, 2, 1)


def kernel(backbone_tokens, params, num_masked_tracks):
    return _run(backbone_tokens, params, num_masked_tracks)
